# baseline (speedup 1.0000x reference)
import functools

import jax
import jax.numpy as jnp
from jax import lax
from jax.experimental import pallas as pl
from jax.experimental.pallas import tpu as pltpu
from jax.experimental.pallas import tpu_sc as plsc

_HID = 16
_NSC = 2
_NSUB = 16
_NW = _NSC * _NSUB
_K = 10
_LANE = 128
_CE = _K * _LANE

_TYPES = ('tasks', 'data', 'devices')
_RELS = {
    0: [('data', 'tasks', 'd2t'), ('tasks', 'data', 't2d'),
        ('tasks', 'devices', 't2dev'), ('devices', 'tasks', 'dev2t'),
        ('data', 'devices', 'd2dev'), ('devices', 'data', 'dev2d'),
        ('tasks', 'tasks', 't2t'), ('tasks', 'tasks', 'tft')],
    1: [('data', 'tasks', 'dmt'), ('tasks', 'data', 'tmd'),
        ('tasks', 'devices', 't2dev'), ('devices', 'tasks', 'dev2t'),
        ('data', 'devices', 'd2dev'), ('devices', 'data', 'dev2d'),
        ('tasks', 'tasks', 't2t'), ('tasks', 'tasks', 'tft')],
}


def _rows_block(n, off=0):
    for r in (1000, 512, 256, 128, 64, 32, 16, 8):
        if n % r == 0 and off % r == 0:
            return r
    return n


def _mm_multi(x, ws):
    n, f = x.shape
    k = len(ws)
    wcat = jnp.concatenate(ws, axis=1)
    r = _rows_block(n)

    def body(x_ref, w_ref, *o_refs):
        y = jnp.dot(x_ref[...], w_ref[...], preferred_element_type=jnp.float32)
        for i, o in enumerate(o_refs):
            o[...] = y[:, _HID * i:_HID * (i + 1)]

    return pl.pallas_call(
        body,
        grid=(n // r,),
        in_specs=[pl.BlockSpec((r, f), lambda i: (i, 0)),
                  pl.BlockSpec((f, _HID * k), lambda i: (0, 0))],
        out_specs=[pl.BlockSpec((r, _HID), lambda i: (i, 0))] * k,
        out_shape=[jax.ShapeDtypeStruct((n, _HID), jnp.float32)] * k,
    )(x, wcat)


def _epi(parts, doff, x_prev, wroot, bsum, g, bln):
    n, f = x_prev.shape
    r = _rows_block(n, doff)
    ob = doff // r

    def body(p_ref, x_ref, wr_ref, bs_ref, g_ref, b_ref, y_ref):
        acc = (p_ref[0] + p_ref[1] + bs_ref[...]
               + jnp.dot(x_ref[...], wr_ref[...],
                         preferred_element_type=jnp.float32))
        m = jnp.mean(acc, axis=-1, keepdims=True)
        v = jnp.mean((acc - m) ** 2, axis=-1, keepdims=True)
        h = (acc - m) / jnp.sqrt(v + 1e-5) * g_ref[...] + b_ref[...]
        y_ref[...] = jnp.where(h >= 0, h, 0.01 * h)

    return pl.pallas_call(
        body,
        grid=(n // r,),
        in_specs=[pl.BlockSpec((2, r, _HID), lambda i: (0, i + ob, 0)),
                  pl.BlockSpec((r, f), lambda i: (i, 0)),
                  pl.BlockSpec((f, _HID), lambda i: (0, 0)),
                  pl.BlockSpec((1, _HID), lambda i: (0, 0)),
                  pl.BlockSpec((1, _HID), lambda i: (0, 0)),
                  pl.BlockSpec((1, _HID), lambda i: (0, 0))],
        out_specs=pl.BlockSpec((r, _HID), lambda i: (i, 0)),
        out_shape=jax.ShapeDtypeStruct((n, _HID), jnp.float32),
    )(parts, x_prev, wroot, bsum, g, bln)


def _sc_scatter(tables, erows, zeros, geom, nacc):
    mesh = plsc.VectorSubcoreMesh(core_axis_name="c", subcore_axis_name="s")
    rps = nacc // _NSUB
    nrel = len(tables)

    @functools.partial(
        pl.kernel,
        out_type=jax.ShapeDtypeStruct((_NSC, nacc, _HID), jnp.float32),
        mesh=mesh,
        scratch_types=[
            pltpu.VMEM((3, 2, _CE), jnp.int32),
            pltpu.VMEM((2, _K, _LANE, _HID), jnp.float32),
            pltpu.VMEM_SHARED((nacc, _HID), jnp.float32),
            pltpu.SemaphoreType.DMA,
            pltpu.SemaphoreType.DMA,
            pltpu.SemaphoreType.DMA,
        ],
        compiler_params=pltpu.CompilerParams(use_tc_tiling_on_sc=False),
    )
    def k(*refs):
        t_refs = refs[0:nrel]
        e_refs = refs[nrel:2 * nrel]
        z_hbm = refs[2 * nrel]
        out_hbm = refs[2 * nrel + 1]
        eidx, rows, acc, isem, gsem, ssem = refs[2 * nrel + 2:]
        c = lax.axis_index("c")
        s = lax.axis_index("s")
        wid = c * _NSUB + s
        pltpu.sync_copy(z_hbm.at[pl.ds(s * rps, rps)],
                        acc.at[pl.ds(s * rps, rps)])
        plsc.subcore_barrier()

        for r in range(nrel):
            t_hbm = t_refs[r]
            e_hbm = e_refs[r]
            nch, doff = geom[r]
            accv = acc.at[pl.ds(doff, nacc - doff)]

            def fire_i(t, e_hbm=e_hbm):
                pltpu.async_copy(e_hbm.at[:, pl.ds(t * _CE, _CE)],
                                 eidx.at[t % 3], isem)

            def drain_i(t, e_hbm=e_hbm):
                pltpu.make_async_copy(e_hbm.at[:, pl.ds(t * _CE, _CE)],
                                      eidx.at[t % 3], isem).wait()

            def fire_g(t, t_hbm=t_hbm):
                for j in range(_K):
                    pltpu.async_copy(t_hbm.at[eidx.at[t % 3, 0, pl.ds(j * _LANE, _LANE)]],
                                     rows.at[t % 2, j], gsem)

            def drain_g(t, t_hbm=t_hbm):
                for j in range(_K):
                    pltpu.make_async_copy(t_hbm.at[eidx.at[t % 3, 0, pl.ds(j * _LANE, _LANE)]],
                                          rows.at[t % 2, j], gsem).wait()

            def fire_s(t, accv=accv):
                for j in range(_K):
                    pltpu.async_copy(rows.at[t % 2, j],
                                     accv.at[eidx.at[t % 3, 1, pl.ds(j * _LANE, _LANE)]], ssem,
                                     add=True)

            def drain_s(t, accv=accv):
                for j in range(_K):
                    pltpu.make_async_copy(rows.at[t % 2, j],
                                          accv.at[eidx.at[t % 3, 1, pl.ds(j * _LANE, _LANE)]],
                                          ssem).wait()

            c0 = (wid * nch) // _NW
            c1 = ((wid + 1) * nch) // _NW

            if nch >= _NW * 4:
                fire_i(c0)
                fire_i(c0 + 1)
                drain_i(c0)
                fire_g(c0)
                drain_g(c0)
                fire_i(c0 + 2)
                drain_i(c0 + 1)
                fire_g(c0 + 1)
                fire_s(c0)

                def chunk(t, carry):
                    drain_g(t)
                    drain_s(t - 1)
                    fire_i(t + 2)
                    drain_i(t + 1)
                    fire_g(t + 1)
                    fire_s(t)
                    return carry

                lax.fori_loop(c0 + 1, c1 - 2, chunk, 0)
                drain_g(c1 - 2)
                drain_s(c1 - 3)
                drain_i(c1 - 1)
                fire_g(c1 - 1)
                fire_s(c1 - 2)
                drain_g(c1 - 1)
                drain_s(c1 - 2)
                fire_s(c1 - 1)
                drain_s(c1 - 1)
            else:
                def simple(t, carry, e_hbm=e_hbm, t_hbm=t_hbm, accv=accv):
                    pltpu.sync_copy(e_hbm.at[:, pl.ds(t * _CE, _CE)],
                                    eidx.at[0])
                    for j in range(_K):
                        pltpu.async_copy(t_hbm.at[eidx.at[0, 0, pl.ds(j * _LANE, _LANE)]],
                                         rows.at[0, j], gsem)
                    for j in range(_K):
                        pltpu.make_async_copy(t_hbm.at[eidx.at[0, 0, pl.ds(j * _LANE, _LANE)]],
                                              rows.at[0, j], gsem).wait()
                    for j in range(_K):
                        pltpu.sync_copy(rows.at[0, j],
                                        accv.at[eidx.at[0, 1, pl.ds(j * _LANE, _LANE)]], add=True)
                    return carry

                lax.fori_loop(c0, c1, simple, 0)

        plsc.subcore_barrier()
        pltpu.sync_copy(acc.at[pl.ds(s * rps, rps)],
                        out_hbm.at[c, pl.ds(s * rps, rps)])

    return k(*tables, *erows, zeros)


def _pad_edges(e, epad, pad_dst):
    pad = epad - e.shape[1]
    if pad == 0:
        return e
    tail = jnp.stack([jnp.zeros((pad,), jnp.int32),
                      jnp.full((pad,), pad_dst, jnp.int32)])
    return jnp.concatenate([e, tail], axis=1)


def kernel(x_tasks, x_data, x_devices, edges, params):
    xs = {'tasks': x_tasks, 'data': x_data, 'devices': x_devices}
    ns = {t: xs[t].shape[0] for t in _TYPES}
    doff = {'tasks': 0, 'data': ns['tasks'],
            'devices': ns['tasks'] + ns['data']}
    ndst = ns['tasks'] + ns['data'] + ns['devices']
    nacc = ((ndst + 1 + _NSUB * 8 - 1) // (_NSUB * 8)) * (_NSUB * 8)
    zeros = jnp.zeros((nacc, _HID), jnp.float32)

    for l in (0, 1):
        rels = _RELS[l]
        lp = params['l' + str(l)]

        by_src = {ty: [r for r, (s, _, _) in enumerate(rels) if s == ty]
                  for ty in _TYPES}
        tables = [None] * len(rels)
        for ty in _TYPES:
            outs = _mm_multi(xs[ty], [lp[rels[r][2]][0] for r in by_src[ty]])
            for pos, r in enumerate(by_src[ty]):
                tables[r] = outs[pos]

        erows, geom = [], []
        for r, (s, d, name) in enumerate(rels):
            e = edges[name]
            ne = e.shape[1]
            epad = ((ne + _CE - 1) // _CE) * _CE
            ep = _pad_edges(e, epad, ndst - doff[d])
            erows.append(ep)
            geom.append((epad // _CE, doff[d]))

        parts = _sc_scatter(tables, erows, zeros, geom, nacc)

        lnp = params['ln']['l' + str(l)]
        nxt = {}
        for ty in _TYPES:
            rel_d = [r for r, (_, d, _) in enumerate(rels) if d == ty]
            wroot = sum(lp[rels[r][2]][2] for r in rel_d)
            bsum = sum(lp[rels[r][2]][1] for r in rel_d).reshape(1, _HID)
            g, bln = lnp[ty]
            nxt[ty] = _epi(parts, doff[ty], xs[ty], wroot, bsum,
                           g.reshape(1, _HID), bln.reshape(1, _HID))
        xs = nxt

    return (xs['tasks'], xs['data'], xs['devices'])


# speedup vs baseline: 1.1228x; 1.1228x over previous
import functools

import jax
import jax.numpy as jnp
from jax import lax
from jax.experimental import pallas as pl
from jax.experimental.pallas import tpu as pltpu
from jax.experimental.pallas import tpu_sc as plsc

_HID = 16
_NSC = 2
_NSUB = 16
_NW = _NSC * _NSUB
_K = 10
_LANE = 128
_CE = _K * _LANE

_TYPES = ('tasks', 'data', 'devices')
_RELS = {
    0: [('data', 'tasks', 'd2t'), ('tasks', 'data', 't2d'),
        ('tasks', 'devices', 't2dev'), ('devices', 'tasks', 'dev2t'),
        ('data', 'devices', 'd2dev'), ('devices', 'data', 'dev2d'),
        ('tasks', 'tasks', 't2t'), ('tasks', 'tasks', 'tft')],
    1: [('data', 'tasks', 'dmt'), ('tasks', 'data', 'tmd'),
        ('tasks', 'devices', 't2dev'), ('devices', 'tasks', 'dev2t'),
        ('data', 'devices', 'd2dev'), ('devices', 'data', 'dev2d'),
        ('tasks', 'tasks', 't2t'), ('tasks', 'tasks', 'tft')],
}


def _rows_block(n, off=0):
    for r in (5000, 2500, 2000, 1000, 512, 256, 128, 64, 32, 16, 8):
        if n % r == 0 and off % r == 0:
            return r
    return n


def _mm_multi(x, ws):
    n, f = x.shape
    k = len(ws)
    wcat = jnp.concatenate(ws, axis=1)
    r = _rows_block(n)

    def body(x_ref, w_ref, *o_refs):
        y = jnp.dot(x_ref[...], w_ref[...], preferred_element_type=jnp.float32)
        for i, o in enumerate(o_refs):
            o[...] = y[:, _HID * i:_HID * (i + 1)]

    return pl.pallas_call(
        body,
        grid=(n // r,),
        in_specs=[pl.BlockSpec((r, f), lambda i: (i, 0)),
                  pl.BlockSpec((f, _HID * k), lambda i: (0, 0))],
        out_specs=[pl.BlockSpec((r, _HID), lambda i: (i, 0))] * k,
        out_shape=[jax.ShapeDtypeStruct((n, _HID), jnp.float32)] * k,
    )(x, wcat)


def _epi(parts, doff, x_prev, wroot, bsum, g, bln):
    n, f = x_prev.shape
    r = _rows_block(n, doff)
    ob = doff // r

    def body(p_ref, x_ref, wr_ref, bs_ref, g_ref, b_ref, y_ref):
        acc = (p_ref[0] + p_ref[1] + bs_ref[...]
               + jnp.dot(x_ref[...], wr_ref[...],
                         preferred_element_type=jnp.float32))
        m = jnp.mean(acc, axis=-1, keepdims=True)
        v = jnp.mean((acc - m) ** 2, axis=-1, keepdims=True)
        h = (acc - m) / jnp.sqrt(v + 1e-5) * g_ref[...] + b_ref[...]
        y_ref[...] = jnp.where(h >= 0, h, 0.01 * h)

    return pl.pallas_call(
        body,
        grid=(n // r,),
        in_specs=[pl.BlockSpec((2, r, _HID), lambda i: (0, i + ob, 0)),
                  pl.BlockSpec((r, f), lambda i: (i, 0)),
                  pl.BlockSpec((f, _HID), lambda i: (0, 0)),
                  pl.BlockSpec((1, _HID), lambda i: (0, 0)),
                  pl.BlockSpec((1, _HID), lambda i: (0, 0)),
                  pl.BlockSpec((1, _HID), lambda i: (0, 0))],
        out_specs=pl.BlockSpec((r, _HID), lambda i: (i, 0)),
        out_shape=jax.ShapeDtypeStruct((n, _HID), jnp.float32),
    )(parts, x_prev, wroot, bsum, g, bln)


def _sc_scatter(tables, erows, zeros, geom, nacc):
    mesh = plsc.VectorSubcoreMesh(core_axis_name="c", subcore_axis_name="s")
    rps = nacc // _NSUB
    nrel = len(tables)

    @functools.partial(
        pl.kernel,
        out_type=jax.ShapeDtypeStruct((_NSC, nacc, _HID), jnp.float32),
        mesh=mesh,
        scratch_types=[
            pltpu.VMEM((3, 2, _CE), jnp.int32),
            pltpu.VMEM((2, _K, _LANE, _HID), jnp.float32),
            pltpu.VMEM_SHARED((nacc, _HID), jnp.float32),
            pltpu.SemaphoreType.DMA,
            pltpu.SemaphoreType.DMA,
            pltpu.SemaphoreType.DMA,
        ],
        compiler_params=pltpu.CompilerParams(use_tc_tiling_on_sc=False),
    )
    def k(*refs):
        t_refs = refs[0:nrel]
        e_refs = refs[nrel:2 * nrel]
        z_hbm = refs[2 * nrel]
        out_hbm = refs[2 * nrel + 1]
        eidx, rows, acc, isem, gsem, ssem = refs[2 * nrel + 2:]
        c = lax.axis_index("c")
        s = lax.axis_index("s")
        wid = c * _NSUB + s
        pltpu.sync_copy(z_hbm.at[pl.ds(s * rps, rps)],
                        acc.at[pl.ds(s * rps, rps)])
        plsc.subcore_barrier()

        for r in range(nrel):
            t_hbm = t_refs[r]
            e_hbm = e_refs[r]
            nch, doff = geom[r]
            accv = acc.at[pl.ds(doff, nacc - doff)]

            def fire_i(t, e_hbm=e_hbm):
                pltpu.async_copy(e_hbm.at[:, pl.ds(t * _CE, _CE)],
                                 eidx.at[t % 3], isem)

            def drain_i(t, e_hbm=e_hbm):
                pltpu.make_async_copy(e_hbm.at[:, pl.ds(t * _CE, _CE)],
                                      eidx.at[t % 3], isem).wait()

            def fire_g(t, t_hbm=t_hbm):
                for j in range(_K):
                    pltpu.async_copy(t_hbm.at[eidx.at[t % 3, 0, pl.ds(j * _LANE, _LANE)]],
                                     rows.at[t % 2, j], gsem)

            def drain_g(t, t_hbm=t_hbm):
                for j in range(_K):
                    pltpu.make_async_copy(t_hbm.at[eidx.at[t % 3, 0, pl.ds(j * _LANE, _LANE)]],
                                          rows.at[t % 2, j], gsem).wait()

            def fire_s(t, accv=accv):
                for j in range(_K):
                    pltpu.async_copy(rows.at[t % 2, j],
                                     accv.at[eidx.at[t % 3, 1, pl.ds(j * _LANE, _LANE)]], ssem,
                                     add=True)

            def drain_s(t, accv=accv):
                for j in range(_K):
                    pltpu.make_async_copy(rows.at[t % 2, j],
                                          accv.at[eidx.at[t % 3, 1, pl.ds(j * _LANE, _LANE)]],
                                          ssem).wait()

            c0 = (wid * nch) // _NW
            c1 = ((wid + 1) * nch) // _NW

            if nch >= _NW * 4:
                fire_i(c0)
                fire_i(c0 + 1)
                drain_i(c0)
                fire_g(c0)
                drain_g(c0)
                fire_i(c0 + 2)
                drain_i(c0 + 1)
                fire_g(c0 + 1)
                fire_s(c0)

                def chunk(t, carry):
                    drain_g(t)
                    drain_s(t - 1)
                    fire_i(t + 2)
                    drain_i(t + 1)
                    fire_g(t + 1)
                    fire_s(t)
                    return carry

                lax.fori_loop(c0 + 1, c1 - 2, chunk, 0)
                drain_g(c1 - 2)
                drain_s(c1 - 3)
                drain_i(c1 - 1)
                fire_g(c1 - 1)
                fire_s(c1 - 2)
                drain_g(c1 - 1)
                drain_s(c1 - 2)
                fire_s(c1 - 1)
                drain_s(c1 - 1)
            else:
                def simple(t, carry, e_hbm=e_hbm, t_hbm=t_hbm, accv=accv):
                    pltpu.sync_copy(e_hbm.at[:, pl.ds(t * _CE, _CE)],
                                    eidx.at[0])
                    for j in range(_K):
                        pltpu.async_copy(t_hbm.at[eidx.at[0, 0, pl.ds(j * _LANE, _LANE)]],
                                         rows.at[0, j], gsem)
                    for j in range(_K):
                        pltpu.make_async_copy(t_hbm.at[eidx.at[0, 0, pl.ds(j * _LANE, _LANE)]],
                                              rows.at[0, j], gsem).wait()
                    for j in range(_K):
                        pltpu.sync_copy(rows.at[0, j],
                                        accv.at[eidx.at[0, 1, pl.ds(j * _LANE, _LANE)]], add=True)
                    return carry

                lax.fori_loop(c0, c1, simple, 0)

        plsc.subcore_barrier()
        pltpu.sync_copy(acc.at[pl.ds(s * rps, rps)],
                        out_hbm.at[c, pl.ds(s * rps, rps)])

    return k(*tables, *erows, zeros)


def _pad_edges(e, epad, pad_dst):
    pad = epad - e.shape[1]
    if pad == 0:
        return e
    tail = jnp.stack([jnp.zeros((pad,), jnp.int32),
                      jnp.full((pad,), pad_dst, jnp.int32)])
    return jnp.concatenate([e, tail], axis=1)


def kernel(x_tasks, x_data, x_devices, edges, params):
    xs = {'tasks': x_tasks, 'data': x_data, 'devices': x_devices}
    ns = {t: xs[t].shape[0] for t in _TYPES}
    doff = {'tasks': 0, 'data': ns['tasks'],
            'devices': ns['tasks'] + ns['data']}
    ndst = ns['tasks'] + ns['data'] + ns['devices']
    nacc = ((ndst + 1 + _NSUB * 8 - 1) // (_NSUB * 8)) * (_NSUB * 8)
    zeros = jnp.zeros((nacc, _HID), jnp.float32)

    for l in (0, 1):
        rels = _RELS[l]
        lp = params['l' + str(l)]

        by_src = {ty: [r for r, (s, _, _) in enumerate(rels) if s == ty]
                  for ty in _TYPES}
        tables = [None] * len(rels)
        for ty in _TYPES:
            outs = _mm_multi(xs[ty], [lp[rels[r][2]][0] for r in by_src[ty]])
            for pos, r in enumerate(by_src[ty]):
                tables[r] = outs[pos]

        erows, geom = [], []
        for r, (s, d, name) in enumerate(rels):
            e = edges[name]
            ne = e.shape[1]
            epad = ((ne + _CE - 1) // _CE) * _CE
            ep = _pad_edges(e, epad, ndst - doff[d])
            erows.append(ep)
            geom.append((epad // _CE, doff[d]))

        parts = _sc_scatter(tables, erows, zeros, geom, nacc)

        lnp = params['ln']['l' + str(l)]
        nxt = {}
        for ty in _TYPES:
            rel_d = [r for r, (_, d, _) in enumerate(rels) if d == ty]
            wroot = sum(lp[rels[r][2]][2] for r in rel_d)
            bsum = sum(lp[rels[r][2]][1] for r in rel_d).reshape(1, _HID)
            g, bln = lnp[ty]
            nxt[ty] = _epi(parts, doff[ty], xs[ty], wroot, bsum,
                           g.reshape(1, _HID), bln.reshape(1, _HID))
        xs = nxt

    return (xs['tasks'], xs['data'], xs['devices'])


# speedup vs baseline: 1.1260x; 1.0028x over previous
import functools

import jax
import jax.numpy as jnp
from jax import lax
from jax.experimental import pallas as pl
from jax.experimental.pallas import tpu as pltpu
from jax.experimental.pallas import tpu_sc as plsc

_HID = 16
_NSC = 2
_NSUB = 16
_NW = _NSC * _NSUB
_K = 10
_LANE = 128
_CE = _K * _LANE

_TYPES = ('tasks', 'data', 'devices')
_RELS = {
    0: [('data', 'tasks', 'd2t'), ('tasks', 'data', 't2d'),
        ('tasks', 'devices', 't2dev'), ('devices', 'tasks', 'dev2t'),
        ('data', 'devices', 'd2dev'), ('devices', 'data', 'dev2d'),
        ('tasks', 'tasks', 't2t'), ('tasks', 'tasks', 'tft')],
    1: [('data', 'tasks', 'dmt'), ('tasks', 'data', 'tmd'),
        ('tasks', 'devices', 't2dev'), ('devices', 'tasks', 'dev2t'),
        ('data', 'devices', 'd2dev'), ('devices', 'data', 'dev2d'),
        ('tasks', 'tasks', 't2t'), ('tasks', 'tasks', 'tft')],
}


def _rows_block(n, off=0):
    for r in (5000, 2500, 2000, 1000, 512, 256, 128, 64, 32, 16, 8):
        if n % r == 0 and off % r == 0:
            return r
    return n


def _mm_multi(x, ws):
    n, f = x.shape
    k = len(ws)
    wcat = jnp.concatenate(ws, axis=1)
    r = _rows_block(n)

    def body(x_ref, w_ref, *o_refs):
        y = jnp.dot(x_ref[...], w_ref[...], preferred_element_type=jnp.float32)
        for i, o in enumerate(o_refs):
            o[...] = y[:, _HID * i:_HID * (i + 1)]

    return pl.pallas_call(
        body,
        grid=(n // r,),
        in_specs=[pl.BlockSpec((r, f), lambda i: (i, 0)),
                  pl.BlockSpec((f, _HID * k), lambda i: (0, 0))],
        out_specs=[pl.BlockSpec((r, _HID), lambda i: (i, 0))] * k,
        out_shape=[jax.ShapeDtypeStruct((n, _HID), jnp.float32)] * k,
    )(x, wcat)


def _epi(parts, doff, x_prev, wroot, bsum, g, bln, wnext=None):
    n, f = x_prev.shape
    r = _rows_block(n, doff)
    ob = doff // r
    k = 0 if wnext is None else wnext.shape[1] // _HID

    def body(p_ref, x_ref, wr_ref, bs_ref, g_ref, b_ref, *rest):
        acc = (p_ref[0] + p_ref[1] + bs_ref[...]
               + jnp.dot(x_ref[...], wr_ref[...],
                         preferred_element_type=jnp.float32))
        m = jnp.mean(acc, axis=-1, keepdims=True)
        v = jnp.mean((acc - m) ** 2, axis=-1, keepdims=True)
        h = (acc - m) / jnp.sqrt(v + 1e-5) * g_ref[...] + b_ref[...]
        y = jnp.where(h >= 0, h, 0.01 * h)
        if k:
            wn_ref, y_ref = rest[0], rest[1]
            p = jnp.dot(y, wn_ref[...], preferred_element_type=jnp.float32)
            for i in range(k):
                rest[2 + i][...] = p[:, _HID * i:_HID * (i + 1)]
        else:
            y_ref = rest[0]
        y_ref[...] = y

    in_specs = [pl.BlockSpec((2, r, _HID), lambda i: (0, i + ob, 0)),
                pl.BlockSpec((r, f), lambda i: (i, 0)),
                pl.BlockSpec((f, _HID), lambda i: (0, 0)),
                pl.BlockSpec((1, _HID), lambda i: (0, 0)),
                pl.BlockSpec((1, _HID), lambda i: (0, 0)),
                pl.BlockSpec((1, _HID), lambda i: (0, 0))]
    args = [parts, x_prev, wroot, bsum, g, bln]
    if k:
        in_specs.append(pl.BlockSpec((_HID, _HID * k), lambda i: (0, 0)))
        args.append(wnext)
    return pl.pallas_call(
        body,
        grid=(n // r,),
        in_specs=in_specs,
        out_specs=[pl.BlockSpec((r, _HID), lambda i: (i, 0))] * (1 + k),
        out_shape=[jax.ShapeDtypeStruct((n, _HID), jnp.float32)] * (1 + k),
    )(*args)


def _sc_scatter(tables, erows, zeros, geom, nacc):
    mesh = plsc.VectorSubcoreMesh(core_axis_name="c", subcore_axis_name="s")
    rps = nacc // _NSUB
    nrel = len(tables)

    @functools.partial(
        pl.kernel,
        out_type=jax.ShapeDtypeStruct((_NSC, nacc, _HID), jnp.float32),
        mesh=mesh,
        scratch_types=[
            pltpu.VMEM((3, 2, _CE), jnp.int32),
            pltpu.VMEM((2, _K, _LANE, _HID), jnp.float32),
            pltpu.VMEM_SHARED((nacc, _HID), jnp.float32),
            pltpu.SemaphoreType.DMA,
            pltpu.SemaphoreType.DMA,
            pltpu.SemaphoreType.DMA,
        ],
        compiler_params=pltpu.CompilerParams(use_tc_tiling_on_sc=False),
    )
    def k(*refs):
        t_refs = refs[0:nrel]
        e_refs = refs[nrel:2 * nrel]
        z_hbm = refs[2 * nrel]
        out_hbm = refs[2 * nrel + 1]
        eidx, rows, acc, isem, gsem, ssem = refs[2 * nrel + 2:]
        c = lax.axis_index("c")
        s = lax.axis_index("s")
        wid = c * _NSUB + s
        pltpu.sync_copy(z_hbm.at[pl.ds(s * rps, rps)],
                        acc.at[pl.ds(s * rps, rps)])
        plsc.subcore_barrier()

        for r in range(nrel):
            t_hbm = t_refs[r]
            e_hbm = e_refs[r]
            nch, doff = geom[r]
            accv = acc.at[pl.ds(doff, nacc - doff)]

            def fire_i(t, e_hbm=e_hbm):
                pltpu.async_copy(e_hbm.at[:, pl.ds(t * _CE, _CE)],
                                 eidx.at[t % 3], isem)

            def drain_i(t, e_hbm=e_hbm):
                pltpu.make_async_copy(e_hbm.at[:, pl.ds(t * _CE, _CE)],
                                      eidx.at[t % 3], isem).wait()

            def fire_g(t, t_hbm=t_hbm):
                for j in range(_K):
                    pltpu.async_copy(t_hbm.at[eidx.at[t % 3, 0, pl.ds(j * _LANE, _LANE)]],
                                     rows.at[t % 2, j], gsem)

            def drain_g(t, t_hbm=t_hbm):
                for j in range(_K):
                    pltpu.make_async_copy(t_hbm.at[eidx.at[t % 3, 0, pl.ds(j * _LANE, _LANE)]],
                                          rows.at[t % 2, j], gsem).wait()

            def fire_s(t, accv=accv):
                for j in range(_K):
                    pltpu.async_copy(rows.at[t % 2, j],
                                     accv.at[eidx.at[t % 3, 1, pl.ds(j * _LANE, _LANE)]], ssem,
                                     add=True)

            def drain_s(t, accv=accv):
                for j in range(_K):
                    pltpu.make_async_copy(rows.at[t % 2, j],
                                          accv.at[eidx.at[t % 3, 1, pl.ds(j * _LANE, _LANE)]],
                                          ssem).wait()

            c0 = (wid * nch) // _NW
            c1 = ((wid + 1) * nch) // _NW

            if nch >= _NW * 4:
                fire_i(c0)
                fire_i(c0 + 1)
                drain_i(c0)
                fire_g(c0)
                drain_g(c0)
                fire_i(c0 + 2)
                drain_i(c0 + 1)
                fire_g(c0 + 1)
                fire_s(c0)

                def chunk(t, carry):
                    drain_g(t)
                    drain_s(t - 1)
                    fire_i(t + 2)
                    drain_i(t + 1)
                    fire_g(t + 1)
                    fire_s(t)
                    return carry

                lax.fori_loop(c0 + 1, c1 - 2, chunk, 0)
                drain_g(c1 - 2)
                drain_s(c1 - 3)
                drain_i(c1 - 1)
                fire_g(c1 - 1)
                fire_s(c1 - 2)
                drain_g(c1 - 1)
                drain_s(c1 - 2)
                fire_s(c1 - 1)
                drain_s(c1 - 1)
            else:
                def simple(t, carry, e_hbm=e_hbm, t_hbm=t_hbm, accv=accv):
                    pltpu.sync_copy(e_hbm.at[:, pl.ds(t * _CE, _CE)],
                                    eidx.at[0])
                    for j in range(_K):
                        pltpu.async_copy(t_hbm.at[eidx.at[0, 0, pl.ds(j * _LANE, _LANE)]],
                                         rows.at[0, j], gsem)
                    for j in range(_K):
                        pltpu.make_async_copy(t_hbm.at[eidx.at[0, 0, pl.ds(j * _LANE, _LANE)]],
                                              rows.at[0, j], gsem).wait()
                    for j in range(_K):
                        pltpu.sync_copy(rows.at[0, j],
                                        accv.at[eidx.at[0, 1, pl.ds(j * _LANE, _LANE)]], add=True)
                    return carry

                lax.fori_loop(c0, c1, simple, 0)

        plsc.subcore_barrier()
        pltpu.sync_copy(acc.at[pl.ds(s * rps, rps)],
                        out_hbm.at[c, pl.ds(s * rps, rps)])

    return k(*tables, *erows, zeros)


def _pad_edges(e, epad, pad_dst):
    pad = epad - e.shape[1]
    if pad == 0:
        return e
    tail = jnp.stack([jnp.zeros((pad,), jnp.int32),
                      jnp.full((pad,), pad_dst, jnp.int32)])
    return jnp.concatenate([e, tail], axis=1)


def kernel(x_tasks, x_data, x_devices, edges, params):
    xs = {'tasks': x_tasks, 'data': x_data, 'devices': x_devices}
    ns = {t: xs[t].shape[0] for t in _TYPES}
    doff = {'tasks': 0, 'data': ns['tasks'],
            'devices': ns['tasks'] + ns['data']}
    ndst = ns['tasks'] + ns['data'] + ns['devices']
    nacc = ((ndst + 1 + _NSUB * 8 - 1) // (_NSUB * 8)) * (_NSUB * 8)
    zeros = jnp.zeros((nacc, _HID), jnp.float32)

    by_src = {ty: [r for r, (s, _, _) in enumerate(_RELS[0]) if s == ty]
              for ty in _TYPES}

    lp = params['l0']
    tables = [None] * len(_RELS[0])
    for ty in _TYPES:
        outs = _mm_multi(xs[ty],
                         [lp[_RELS[0][r][2]][0] for r in by_src[ty]])
        for pos, r in enumerate(by_src[ty]):
            tables[r] = outs[pos]

    for l in (0, 1):
        rels = _RELS[l]
        lp = params['l' + str(l)]

        erows, geom = [], []
        for r, (s, d, name) in enumerate(rels):
            e = edges[name]
            ne = e.shape[1]
            epad = ((ne + _CE - 1) // _CE) * _CE
            ep = _pad_edges(e, epad, ndst - doff[d])
            erows.append(ep)
            geom.append((epad // _CE, doff[d]))

        parts = _sc_scatter(tables, erows, zeros, geom, nacc)

        lnp = params['ln']['l' + str(l)]
        nrels = _RELS[1] if l == 0 else None
        nlp = params['l1'] if l == 0 else None
        nxt = {}
        tables = [None] * len(rels)
        for ty in _TYPES:
            rel_d = [r for r, (_, d, _) in enumerate(rels) if d == ty]
            wroot = sum(lp[rels[r][2]][2] for r in rel_d)
            bsum = sum(lp[rels[r][2]][1] for r in rel_d).reshape(1, _HID)
            g, bln = lnp[ty]
            wnext = None
            if l == 0:
                wnext = jnp.concatenate(
                    [nlp[nrels[r][2]][0] for r in by_src[ty]], axis=1)
            outs = _epi(parts, doff[ty], xs[ty], wroot, bsum,
                        g.reshape(1, _HID), bln.reshape(1, _HID), wnext)
            nxt[ty] = outs[0]
            if l == 0:
                for pos, r in enumerate(by_src[ty]):
                    tables[r] = outs[1 + pos]
        xs = nxt

    return (xs['tasks'], xs['data'], xs['devices'])
